# initial kernel scaffold (unmeasured)
import jax
import jax.numpy as jnp
from jax import lax
from jax.experimental import pallas as pl
from jax.experimental.pallas import tpu as pltpu

N_DEV = 4
M_SH = 2048
K = 8192
N = 4096
N_PER = N // N_DEV
MH = M_SH // 2
KT = 1024
NKT = K // KT


def kernel(x, w_mat):
    def body(x_hbm, w_hbm, out_hbm,
             x_bf, w_stage, w_bf, acc, stage, send_buf,
             recv_buf, copy_sem, send_sem, recv_sems):
        my = lax.axis_index("i")

        def local_copy(src, dst):
            cp = pltpu.make_async_copy(src, dst, copy_sem)
            cp.start()
            cp.wait()

        for h in range(2):
            for kt in range(NKT):
                local_copy(x_hbm.at[pl.ds(h * MH, MH), pl.ds(kt * KT, KT)],
                           stage)
                x_bf[:, pl.ds(kt * KT, KT)] = stage[...].astype(jnp.bfloat16)

            for t in (1, 2, 3, 0):
                j = (my + t) % N_DEV
                for kt in range(NKT):
                    local_copy(
                        w_hbm.at[pl.ds(kt * KT, KT), pl.ds(j * N_PER, N_PER)],
                        w_stage)
                    w_bf[...] = w_stage[...].astype(jnp.bfloat16)
                    part = jnp.dot(x_bf[:, pl.ds(kt * KT, KT)], w_bf[...],
                                   preferred_element_type=jnp.float32)
                    if kt == 0:
                        acc[...] = part
                    else:
                        acc[...] = acc[...] + part
                y = jnp.maximum(acc[...], 0.0)
                if t == 0:
                    stage[...] = y
                    local_copy(stage,
                               out_hbm.at[pl.ds(my * M_SH + h * MH, MH), :])
                else:
                    send_buf[...] = y.astype(jnp.bfloat16)
                    rdma = pltpu.make_async_remote_copy(
                        src_ref=send_buf,
                        dst_ref=recv_buf.at[my, h],
                        send_sem=send_sem,
                        recv_sem=recv_sems.at[my, h],
                        device_id=(j,),
                        device_id_type=pl.DeviceIdType.MESH,
                    )
                    rdma.start()
                    rdma.wait_send()

        for h in range(2):
            for t in (3, 2, 1):
                s = (my + t) % N_DEV
                recv = pltpu.make_async_remote_copy(
                    src_ref=send_buf,
                    dst_ref=recv_buf.at[s, h],
                    send_sem=send_sem,
                    recv_sem=recv_sems.at[s, h],
                    device_id=(my,),
                    device_id_type=pl.DeviceIdType.MESH,
                )
                recv.wait_recv()
                stage[...] = recv_buf[s, h].astype(jnp.float32)
                local_copy(stage,
                           out_hbm.at[pl.ds(s * M_SH + h * MH, MH), :])

    return pl.pallas_call(
        body,
        out_shape=jax.ShapeDtypeStruct((N_DEV * M_SH, N_PER), jnp.float32),
        in_specs=[
            pl.BlockSpec(memory_space=pltpu.ANY),
            pl.BlockSpec(memory_space=pltpu.ANY),
        ],
        out_specs=pl.BlockSpec(memory_space=pltpu.ANY),
        scratch_shapes=[
            pltpu.VMEM((MH, K), jnp.bfloat16),
            pltpu.VMEM((KT, N_PER), jnp.float32),
            pltpu.VMEM((KT, N_PER), jnp.bfloat16),
            pltpu.VMEM((MH, N_PER), jnp.float32),
            pltpu.VMEM((MH, N_PER), jnp.float32),
            pltpu.VMEM((MH, N_PER), jnp.bfloat16),
            pltpu.VMEM((N_DEV, 2, MH, N_PER), jnp.bfloat16),
            pltpu.SemaphoreType.DMA,
            pltpu.SemaphoreType.DMA,
            pltpu.SemaphoreType.DMA((N_DEV, 2)),
        ],
        compiler_params=pltpu.CompilerParams(collective_id=0),
    )(x, w_mat)


# baseline (device time: 636209 ns/iter reference)
import jax
import jax.numpy as jnp
from jax import lax
from jax.experimental import pallas as pl
from jax.experimental.pallas import tpu as pltpu

N_DEV = 4
M_SH = 2048
K = 8192
N = 4096
N_PER = N // N_DEV
MH = M_SH // 2
KT = 512
NKT = K // KT
XT = 512
NXT = K // XT


def kernel(x, w_mat):
    def body(x_hbm, w_hbm, out_hbm, recv_hbm,
             x_bf, w_stage, w_bf, acc, x_stage, send_buf,
             copy_sem, send_sem, recv_sems):
        my = lax.axis_index("i")

        def local_copy(src, dst):
            cp = pltpu.make_async_copy(src, dst, copy_sem)
            cp.start()
            cp.wait()

        for h in range(2):
            def xconv(kt, carry):
                local_copy(x_hbm.at[pl.ds(h * MH, MH), pl.ds(kt * XT, XT)],
                           x_stage)
                x_bf[:, pl.ds(kt * XT, XT)] = x_stage[...].astype(jnp.bfloat16)
                return carry

            lax.fori_loop(0, NXT, xconv, 0)

            for t in (1, 2, 3, 0):
                j = (my + t) % N_DEV
                acc[...] = jnp.zeros((MH, N_PER), jnp.float32)

                def kstep(kt, carry):
                    local_copy(
                        w_hbm.at[pl.ds(kt * KT, KT), pl.ds(j * N_PER, N_PER)],
                        w_stage)
                    w_bf[...] = w_stage[...].astype(jnp.bfloat16)
                    acc[...] = acc[...] + jnp.dot(
                        x_bf[:, pl.ds(kt * KT, KT)], w_bf[...],
                        preferred_element_type=jnp.float32)
                    return carry

                lax.fori_loop(0, NKT, kstep, 0)

                if t == 0:
                    acc[...] = jnp.maximum(acc[...], 0.0)
                    local_copy(acc,
                               out_hbm.at[pl.ds(my * M_SH + h * MH, MH), :])
                else:
                    send_buf[...] = jnp.maximum(acc[...], 0.0).astype(
                        jnp.bfloat16)
                    rdma = pltpu.make_async_remote_copy(
                        src_ref=send_buf,
                        dst_ref=recv_hbm.at[3 - t, h],
                        send_sem=send_sem,
                        recv_sem=recv_sems.at[3 - t, h],
                        device_id=(j,),
                        device_id_type=pl.DeviceIdType.MESH,
                    )
                    rdma.start()
                    rdma.wait_send()

        for h in range(2):
            for u in (3, 2, 1):
                s = (my + u) % N_DEV
                recv = pltpu.make_async_remote_copy(
                    src_ref=send_buf,
                    dst_ref=recv_hbm.at[u - 1, h],
                    send_sem=send_sem,
                    recv_sem=recv_sems.at[u - 1, h],
                    device_id=(my,),
                    device_id_type=pl.DeviceIdType.MESH,
                )
                recv.wait_recv()
                local_copy(recv_hbm.at[u - 1, h], send_buf)
                acc[...] = send_buf[...].astype(jnp.float32)
                local_copy(acc,
                           out_hbm.at[pl.ds(s * M_SH + h * MH, MH), :])

    out, _ = pl.pallas_call(
        body,
        out_shape=[
            jax.ShapeDtypeStruct((N_DEV * M_SH, N_PER), jnp.float32),
            jax.ShapeDtypeStruct((3, 2, MH, N_PER), jnp.bfloat16),
        ],
        in_specs=[
            pl.BlockSpec(memory_space=pltpu.MemorySpace.HBM),
            pl.BlockSpec(memory_space=pltpu.MemorySpace.HBM),
        ],
        out_specs=[
            pl.BlockSpec(memory_space=pltpu.MemorySpace.HBM),
            pl.BlockSpec(memory_space=pltpu.MemorySpace.HBM),
        ],
        scratch_shapes=[
            pltpu.VMEM((MH, K), jnp.bfloat16),
            pltpu.VMEM((KT, N_PER), jnp.float32),
            pltpu.VMEM((KT, N_PER), jnp.bfloat16),
            pltpu.VMEM((MH, N_PER), jnp.float32),
            pltpu.VMEM((MH, XT), jnp.float32),
            pltpu.VMEM((MH, N_PER), jnp.bfloat16),
            pltpu.SemaphoreType.DMA,
            pltpu.SemaphoreType.DMA,
            pltpu.SemaphoreType.DMA((3, 2)),
        ],
    )(x, w_mat)
    return out


# device time: 312513 ns/iter; 2.0358x vs baseline; 2.0358x over previous
import jax
import jax.numpy as jnp
from jax import lax
from jax.experimental import pallas as pl
from jax.experimental.pallas import tpu as pltpu

N_DEV = 4
M_SH = 2048
K = 8192
N = 4096
N_PER = N // N_DEV
MH = M_SH // 2
KT = 512
NKT = K // KT
XT = 256
NXT = K // XT


def kernel(x, w_mat):
    def body(x_hbm, w_hbm, out_hbm, recv_hbm,
             x_bf, w_stage, w_bf, acc, x_stage, send_buf,
             copy_sem, wsems, xsems, send_sems, recv_sems):
        my = lax.axis_index("i")

        def local_copy(src, dst):
            cp = pltpu.make_async_copy(src, dst, copy_sem)
            cp.start()
            cp.wait()

        pending = {0: None, 1: None}

        for h in range(2):
            def xcp(kt, slot):
                return pltpu.make_async_copy(
                    x_hbm.at[pl.ds(h * MH, MH), pl.ds(kt * XT, XT)],
                    x_stage.at[slot], xsems.at[slot])

            xcp(0, 0).start()

            def xconv(kt, carry):
                slot = lax.rem(kt, 2)

                @pl.when(kt + 1 < NXT)
                def _():
                    xcp(kt + 1, 1 - slot).start()

                xcp(kt, slot).wait()
                x_bf[:, pl.ds(kt * XT, XT)] = (
                    x_stage[slot].astype(jnp.bfloat16))
                return carry

            lax.fori_loop(0, NXT, xconv, 0)

            for t in (1, 2, 3, 0):
                j = (my + t) % N_DEV
                col = pl.ds(j * N_PER, N_PER)
                acc[...] = jnp.zeros((MH, N_PER), jnp.float32)

                def wcp(kt, slot, col=col):
                    return pltpu.make_async_copy(
                        w_hbm.at[pl.ds(kt * KT, KT), col],
                        w_stage.at[slot], wsems.at[slot])

                wcp(0, 0).start()

                def kstep(kt, carry, wcp=wcp):
                    slot = lax.rem(kt, 2)

                    @pl.when(kt + 1 < NKT)
                    def _():
                        wcp(kt + 1, 1 - slot).start()

                    wcp(kt, slot).wait()
                    w_bf[...] = w_stage[slot].astype(jnp.bfloat16)
                    acc[...] = acc[...] + jnp.dot(
                        x_bf[:, pl.ds(kt * KT, KT)], w_bf[...],
                        preferred_element_type=jnp.float32)
                    return carry

                lax.fori_loop(0, NKT, kstep, 0)

                if t == 0:
                    acc[...] = jnp.maximum(acc[...], 0.0)
                    local_copy(acc,
                               out_hbm.at[pl.ds(my * M_SH + h * MH, MH), :])
                else:
                    s = (t - 1) % 2
                    if pending[s] is not None:
                        pending[s].wait_send()
                    send_buf[s] = jnp.maximum(acc[...], 0.0).astype(
                        jnp.bfloat16)
                    rdma = pltpu.make_async_remote_copy(
                        src_ref=send_buf.at[s],
                        dst_ref=recv_hbm.at[3 - t, h],
                        send_sem=send_sems.at[s],
                        recv_sem=recv_sems.at[3 - t, h],
                        device_id=(j,),
                        device_id_type=pl.DeviceIdType.MESH,
                    )
                    rdma.start()
                    pending[s] = rdma

        for s in (0, 1):
            if pending[s] is not None:
                pending[s].wait_send()
                pending[s] = None

        for h in range(2):
            for u in (3, 2, 1):
                s = (my + u) % N_DEV
                recv = pltpu.make_async_remote_copy(
                    src_ref=send_buf.at[0],
                    dst_ref=recv_hbm.at[u - 1, h],
                    send_sem=send_sems.at[0],
                    recv_sem=recv_sems.at[u - 1, h],
                    device_id=(my,),
                    device_id_type=pl.DeviceIdType.MESH,
                )
                recv.wait_recv()
                local_copy(recv_hbm.at[u - 1, h], send_buf.at[0])
                acc[...] = send_buf[0].astype(jnp.float32)
                local_copy(acc,
                           out_hbm.at[pl.ds(s * M_SH + h * MH, MH), :])

    out, _ = pl.pallas_call(
        body,
        out_shape=[
            jax.ShapeDtypeStruct((N_DEV * M_SH, N_PER), jnp.float32),
            jax.ShapeDtypeStruct((3, 2, MH, N_PER), jnp.bfloat16),
        ],
        in_specs=[
            pl.BlockSpec(memory_space=pltpu.MemorySpace.HBM),
            pl.BlockSpec(memory_space=pltpu.MemorySpace.HBM),
        ],
        out_specs=[
            pl.BlockSpec(memory_space=pltpu.MemorySpace.HBM),
            pl.BlockSpec(memory_space=pltpu.MemorySpace.HBM),
        ],
        scratch_shapes=[
            pltpu.VMEM((MH, K), jnp.bfloat16),
            pltpu.VMEM((2, KT, N_PER), jnp.float32),
            pltpu.VMEM((KT, N_PER), jnp.bfloat16),
            pltpu.VMEM((MH, N_PER), jnp.float32),
            pltpu.VMEM((2, MH, XT), jnp.float32),
            pltpu.VMEM((2, MH, N_PER), jnp.bfloat16),
            pltpu.SemaphoreType.DMA,
            pltpu.SemaphoreType.DMA((2,)),
            pltpu.SemaphoreType.DMA((2,)),
            pltpu.SemaphoreType.DMA((2,)),
            pltpu.SemaphoreType.DMA((3, 2)),
        ],
    )(x, w_mat)
    return out
